# Initial kernel scaffold; baseline (speedup 1.0000x reference)
import jax
import jax.numpy as jnp
from jax.experimental import pallas as pl


def kernel(x, W1, b1, g1, be1, W2, b2, g2, be2, W3, b3, g3, be3, Wg1, bg1, gg1, beg1, Wg2, bg2, gg2, beg2, W4, b4, g4, be4):
    raise NotImplementedError("write your pallas kernel here")


# speedup vs baseline: 4.4120x; 4.4120x over previous
import functools
import jax
import jax.numpy as jnp
from jax.experimental import pallas as pl

EPS = 1e-5
NEG = -1e30
DEF = jax.lax.Precision.DEFAULT
HI = jax.lax.Precision.HIGHEST


def _dot(a, b, dims, prec):
    return jax.lax.dot_general(a, b, (dims, ((), ())),
                               preferred_element_type=jnp.float32,
                               precision=prec)


def _onehot_iter(dcur, iota):
    t = jnp.max(dcur, axis=1, keepdims=True)
    sel = jnp.where(dcur >= t, iota, jnp.int32(2**30))
    ii = jnp.min(sel, axis=1, keepdims=True)
    oh = iota == ii
    return oh, jnp.where(oh, NEG, dcur)


def _knn_cov_kernel(x_ref, xt_ref, w1_ref, b1_ref, y1_ref, st_ref):
    b = pl.program_id(0)
    q = pl.program_id(1)
    Q = y1_ref.shape[1]
    xb = x_ref[0]
    xt = xt_ref[0]
    xq = xt_ref[0, pl.ds(q * Q, Q), :]

    xx = jnp.sum(xb * xb, axis=0, keepdims=True)
    nq = jnp.sum(xq * xq, axis=1, keepdims=True)
    dd = _dot(xq, xt, ((1,), (1,)), DEF)
    d = (-nq + 2.0 * dd) - xx

    xt1 = xt.astype(jnp.bfloat16).astype(jnp.float32)
    r1 = xt - xt1
    xt2 = r1.astype(jnp.bfloat16).astype(jnp.float32)
    xt3c = jnp.concatenate([xt1, xt2, r1 - xt2], axis=1)
    iota = jax.lax.broadcasted_iota(jnp.int32, d.shape, 1)
    dcur = d
    ck = []
    for _ in range(16):
        oh, dcur = _onehot_iter(dcur, iota)
        g = _dot(oh.astype(jnp.float32), xt3c, ((1,), (0,)), HI)
        ck.append((g[:, 0:3] + g[:, 3:6]) + g[:, 6:9])
    mu = ck[0]
    for k in range(1, 16):
        mu = mu + ck[k]
    mu = mu * (1.0 / 16.0)
    covs = [jnp.zeros((Q, 1), jnp.float32) for _ in range(9)]
    for k in range(16):
        e = (ck[k] - mu).astype(jnp.bfloat16).astype(jnp.float32)
        for i in range(3):
            for j in range(3):
                covs[3 * i + j] += e[:, i:i + 1] * e[:, j:j + 1]
    h0 = jnp.concatenate([xq] + covs, axis=1)

    y = _dot(h0, w1_ref[...], ((1,), (0,)), DEF) + b1_ref[...]
    y1_ref[0] = y
    @pl.when(jnp.logical_and(b == 0, q == 0))
    def _():
        st_ref[...] = jnp.zeros_like(st_ref)
    st_ref[0:1, :] += jnp.sum(y, axis=0, keepdims=True)
    st_ref[1:2, :] += jnp.sum(y * y, axis=0, keepdims=True)


def _run_knn_cov(x, xt, w1t, b1, Q=256):
    B, N, _ = xt.shape
    C = w1t.shape[1]
    grid = (B, N // Q)
    y1, st = pl.pallas_call(
        _knn_cov_kernel,
        grid=grid,
        in_specs=[
            pl.BlockSpec((1, 3, N), lambda b, q: (b, 0, 0)),
            pl.BlockSpec((1, N, 3), lambda b, q: (b, 0, 0)),
            pl.BlockSpec((12, C), lambda b, q: (0, 0)),
            pl.BlockSpec((1, C), lambda b, q: (0, 0)),
        ],
        out_specs=[
            pl.BlockSpec((1, Q, C), lambda b, q: (b, q, 0)),
            pl.BlockSpec((2, C), lambda b, q: (0, 0)),
        ],
        out_shape=[
            jax.ShapeDtypeStruct((B, N, C), jnp.float32),
            jax.ShapeDtypeStruct((2, C), jnp.float32),
        ],
    )(x, xt, w1t, b1)
    return y1, st


def _conv_bn_kernel(y_ref, sc_ref, sh_ref, w_ref, b_ref, o_ref, st_ref):
    b = pl.program_id(0)
    q = pl.program_id(1)
    h = jnp.maximum(y_ref[0] * sc_ref[...] + sh_ref[...], 0.0)
    y = _dot(h, w_ref[...], ((1,), (0,)), DEF) + b_ref[...]
    o_ref[0] = y
    @pl.when(jnp.logical_and(b == 0, q == 0))
    def _():
        st_ref[...] = jnp.zeros_like(st_ref)
    st_ref[0:1, :] += jnp.sum(y, axis=0, keepdims=True)
    st_ref[1:2, :] += jnp.sum(y * y, axis=0, keepdims=True)


def _run_conv_bn(yprev, scale, shift, wt, bias, Q=256):
    B, N, Cin = yprev.shape
    Cout = wt.shape[1]
    grid = (B, N // Q)
    y, st = pl.pallas_call(
        _conv_bn_kernel,
        grid=grid,
        in_specs=[
            pl.BlockSpec((1, Q, Cin), lambda b, q: (b, q, 0)),
            pl.BlockSpec((1, Cin), lambda b, q: (0, 0)),
            pl.BlockSpec((1, Cin), lambda b, q: (0, 0)),
            pl.BlockSpec((Cin, Cout), lambda b, q: (0, 0)),
            pl.BlockSpec((1, Cout), lambda b, q: (0, 0)),
        ],
        out_specs=[
            pl.BlockSpec((1, Q, Cout), lambda b, q: (b, q, 0)),
            pl.BlockSpec((2, Cout), lambda b, q: (0, 0)),
        ],
        out_shape=[
            jax.ShapeDtypeStruct((B, N, Cout), jnp.float32),
            jax.ShapeDtypeStruct((2, Cout), jnp.float32),
        ],
    )(yprev, scale, shift, wt, bias)
    return y, st


def _graph_kernel(y_ref, yt_ref, sc_ref, sh_ref, sct_ref, sht_ref,
                  w_ref, b_ref, o_ref, st_ref):
    b = pl.program_id(0)
    q = pl.program_id(1)
    Q = o_ref.shape[1]
    C = y_ref.shape[2]
    h = jnp.maximum(y_ref[0] * sc_ref[...] + sh_ref[...], 0.0)
    ht = jnp.maximum(yt_ref[0] * sct_ref[...] + sht_ref[...], 0.0)
    hq = jnp.maximum(y_ref[0, pl.ds(q * Q, Q), :] * sc_ref[...]
                     + sh_ref[...], 0.0)

    nq = jnp.sum(hq * hq, axis=1, keepdims=True)
    nm = jnp.sum(ht * ht, axis=0, keepdims=True)
    dd = _dot(hq, h, ((1,), (1,)), DEF)
    d = (-nq + 2.0 * dd) - nm

    iota = jax.lax.broadcasted_iota(jnp.int32, d.shape, 1)
    acc = jnp.full((Q, C), NEG, jnp.float32)
    dcur = d
    for _ in range(16):
        oh, dcur = _onehot_iter(dcur, iota)
        feat = _dot(oh.astype(jnp.float32), h, ((1,), (0,)), HI)
        acc = jnp.maximum(acc, feat)
    y = _dot(acc, w_ref[...], ((1,), (0,)), DEF) + b_ref[...]
    o_ref[0] = y
    @pl.when(jnp.logical_and(b == 0, q == 0))
    def _():
        st_ref[...] = jnp.zeros_like(st_ref)
    st_ref[0:1, :] += jnp.sum(y, axis=0, keepdims=True)
    st_ref[1:2, :] += jnp.sum(y * y, axis=0, keepdims=True)


def _run_graph(yprev, yprevT, scale, shift, wt, bias, Q=128):
    B, N, Cin = yprev.shape
    Cout = wt.shape[1]
    grid = (B, N // Q)
    y, st = pl.pallas_call(
        _graph_kernel,
        grid=grid,
        in_specs=[
            pl.BlockSpec((1, N, Cin), lambda b, q: (b, 0, 0)),
            pl.BlockSpec((1, Cin, N), lambda b, q: (b, 0, 0)),
            pl.BlockSpec((1, Cin), lambda b, q: (0, 0)),
            pl.BlockSpec((1, Cin), lambda b, q: (0, 0)),
            pl.BlockSpec((Cin, 1), lambda b, q: (0, 0)),
            pl.BlockSpec((Cin, 1), lambda b, q: (0, 0)),
            pl.BlockSpec((Cin, Cout), lambda b, q: (0, 0)),
            pl.BlockSpec((1, Cout), lambda b, q: (0, 0)),
        ],
        out_specs=[
            pl.BlockSpec((1, Q, Cout), lambda b, q: (b, q, 0)),
            pl.BlockSpec((2, Cout), lambda b, q: (0, 0)),
        ],
        out_shape=[
            jax.ShapeDtypeStruct((B, N, Cout), jnp.float32),
            jax.ShapeDtypeStruct((2, Cout), jnp.float32),
        ],
    )(yprev, yprevT, scale, shift, scale.T, shift.T, wt, bias)
    return y, st


def _final_kernel(y_ref, sc_ref, sh_ref, w_ref, b_ref,
                  mx_ref, mn_ref, st_ref):
    b = pl.program_id(0)
    q = pl.program_id(1)
    h = jnp.maximum(y_ref[0] * sc_ref[...] + sh_ref[...], 0.0)
    y = _dot(h, w_ref[...], ((1,), (0,)), DEF) + b_ref[...]
    @pl.when(jnp.logical_and(b == 0, q == 0))
    def _():
        st_ref[...] = jnp.zeros_like(st_ref)
    st_ref[0:1, :] += jnp.sum(y, axis=0, keepdims=True)
    st_ref[1:2, :] += jnp.sum(y * y, axis=0, keepdims=True)
    @pl.when(q == 0)
    def _():
        mx_ref[...] = jnp.full_like(mx_ref, NEG)
        mn_ref[...] = jnp.full_like(mn_ref, -NEG)
    mx_ref[0] = jnp.maximum(mx_ref[0], jnp.max(y, axis=0, keepdims=True))
    mn_ref[0] = jnp.minimum(mn_ref[0], jnp.min(y, axis=0, keepdims=True))


def _run_final(yprev, scale, shift, wt, bias, Q=256):
    B, N, Cin = yprev.shape
    Cout = wt.shape[1]
    grid = (B, N // Q)
    mx, mn, st = pl.pallas_call(
        _final_kernel,
        grid=grid,
        in_specs=[
            pl.BlockSpec((1, Q, Cin), lambda b, q: (b, q, 0)),
            pl.BlockSpec((1, Cin), lambda b, q: (0, 0)),
            pl.BlockSpec((1, Cin), lambda b, q: (0, 0)),
            pl.BlockSpec((Cin, Cout), lambda b, q: (0, 0)),
            pl.BlockSpec((1, Cout), lambda b, q: (0, 0)),
        ],
        out_specs=[
            pl.BlockSpec((1, 1, Cout), lambda b, q: (b, 0, 0)),
            pl.BlockSpec((1, 1, Cout), lambda b, q: (b, 0, 0)),
            pl.BlockSpec((2, Cout), lambda b, q: (0, 0)),
        ],
        out_shape=[
            jax.ShapeDtypeStruct((B, 1, Cout), jnp.float32),
            jax.ShapeDtypeStruct((B, 1, Cout), jnp.float32),
            jax.ShapeDtypeStruct((2, Cout), jnp.float32),
        ],
    )(yprev, scale, shift, wt, bias)
    return mx[:, 0], mn[:, 0], st


def _bn_coeffs(st, count, g, be):
    mean = st[0] / count
    var = st[1] / count - mean * mean
    isg = g / jnp.sqrt(var + EPS)
    scale = isg[None, :]
    shift = (be - mean * isg)[None, :]
    return scale, shift


def kernel(x, W1, b1, g1, be1, W2, b2, g2, be2, W3, b3, g3, be3,
           Wg1, bg1, gg1, beg1, Wg2, bg2, gg2, beg2, W4, b4, g4, be4):
    B, C, N = x.shape
    cnt = B * N
    xt = jnp.transpose(x, (0, 2, 1))

    y1, st1 = _run_knn_cov(x, xt, W1.T, b1[None, :])
    sc1, sh1 = _bn_coeffs(st1, cnt, g1, be1)
    y2, st2 = _run_conv_bn(y1, sc1, sh1, W2.T, b2[None, :])
    sc2, sh2 = _bn_coeffs(st2, cnt, g2, be2)
    y3, st3 = _run_conv_bn(y2, sc2, sh2, W3.T, b3[None, :])
    sc3, sh3 = _bn_coeffs(st3, cnt, g3, be3)
    y3T = jnp.transpose(y3, (0, 2, 1))
    y4, st4 = _run_graph(y3, y3T, sc3, sh3, Wg1.T, bg1[None, :])
    sc4, sh4 = _bn_coeffs(st4, cnt, gg1, beg1)
    y4T = jnp.transpose(y4, (0, 2, 1))
    y5, st5 = _run_graph(y4, y4T, sc4, sh4, Wg2.T, bg2[None, :])
    sc5, sh5 = _bn_coeffs(st5, cnt, gg2, beg2)
    mx, mn, st6 = _run_final(y5, sc5, sh5, W4.T, b4[None, :])
    sc6, sh6 = _bn_coeffs(st6, cnt, g4, be4)
    out = jnp.where(sc6 > 0, mx * sc6, mn * sc6) + sh6
    return out
